# baseline (speedup 1.0000x reference)
import functools

import jax
import jax.numpy as jnp
from jax import lax
from jax.experimental import pallas as pl
from jax.experimental.pallas import tpu as pltpu
from jax.experimental.pallas import tpu_sc as plsc

_L = 16


def _lookup_body(y_hbm, table_hbm, out_hbm, y_v, win_v, out_v):
    is_lead = jnp.logical_and(
        lax.axis_index("c") == 0, lax.axis_index("s") == 0
    )

    @pl.when(is_lead)
    def _():
        pltpu.sync_copy(y_hbm, y_v.at[pl.ds(0, 8)])
        y = y_v[...][0]
        n = table_hbm.shape[0]
        start = jnp.minimum((y // 8) * 8, n - _L)
        start = pl.multiple_of(start, 8)
        pltpu.sync_copy(table_hbm.at[pl.ds(start, _L)], win_v)
        win = win_v[...]
        off = jnp.full((_L, 1), y - start, jnp.int32)
        out_v[...] = lax.gather(
            win,
            off,
            lax.GatherDimensionNumbers(
                offset_dims=(),
                collapsed_slice_dims=(0,),
                start_index_map=(0,),
            ),
            slice_sizes=(1,),
            mode=lax.GatherScatterMode.PROMISE_IN_BOUNDS,
        )
        pltpu.sync_copy(out_v.at[pl.ds(0, 8)], out_hbm)


_mesh = plsc.VectorSubcoreMesh(core_axis_name="c", subcore_axis_name="s")

_lookup = functools.partial(
    pl.kernel,
    mesh=_mesh,
    out_type=jax.ShapeDtypeStruct((8,), jnp.int32),
    scratch_types=[
        pltpu.VMEM((_L,), jnp.int32),
        pltpu.VMEM((_L,), jnp.int32),
        pltpu.VMEM((_L,), jnp.int32),
    ],
)(_lookup_body)


def kernel(y, label_permutation):
    out_dtype = label_permutation.dtype
    table32 = label_permutation.astype(jnp.int32)
    y32 = jnp.full((8,), y, jnp.int32)
    out = _lookup(y32, table32)
    return out[0].astype(out_dtype)


# speedup vs baseline: 1.0892x; 1.0892x over previous
import functools

import jax
import jax.numpy as jnp
from jax import lax
from jax.experimental import pallas as pl
from jax.experimental.pallas import tpu as pltpu
from jax.experimental.pallas import tpu_sc as plsc

_L = 16


def _lookup_body(y_hbm, table_hbm, out_hbm, y_v, win_v, out_v):
    pltpu.sync_copy(y_hbm, y_v.at[pl.ds(0, 8)])
    y = y_v[...][0]
    n = table_hbm.shape[0]
    start = jnp.minimum((y // 8) * 8, n - _L)
    start = pl.multiple_of(start, 8)
    pltpu.sync_copy(table_hbm.at[pl.ds(start, _L)], win_v)
    win = win_v[...]
    off = jnp.full((_L, 1), y - start, jnp.int32)
    out_v[...] = lax.gather(
        win,
        off,
        lax.GatherDimensionNumbers(
            offset_dims=(),
            collapsed_slice_dims=(0,),
            start_index_map=(0,),
        ),
        slice_sizes=(1,),
        mode=lax.GatherScatterMode.PROMISE_IN_BOUNDS,
    )
    pltpu.sync_copy(out_v.at[pl.ds(0, 8)], out_hbm)


_mesh = plsc.VectorSubcoreMesh(
    core_axis_name="c", subcore_axis_name="s", num_cores=1, num_subcores=1
)

_lookup = functools.partial(
    pl.kernel,
    mesh=_mesh,
    out_type=jax.ShapeDtypeStruct((8,), jnp.int32),
    scratch_types=[
        pltpu.VMEM((_L,), jnp.int32),
        pltpu.VMEM((_L,), jnp.int32),
        pltpu.VMEM((_L,), jnp.int32),
    ],
)(_lookup_body)


def kernel(y, label_permutation):
    out_dtype = label_permutation.dtype
    table32 = label_permutation.astype(jnp.int32)
    y32 = jnp.full((8,), y, jnp.int32)
    out = _lookup(y32, table32)
    return out[0].astype(out_dtype)


# speedup vs baseline: 1.0904x; 1.0012x over previous
import functools

import jax
import jax.numpy as jnp
from jax.experimental import pallas as pl
from jax.experimental.pallas import tpu as pltpu
from jax.experimental.pallas import tpu_sc as plsc

_W = 8


def _lookup_body(y_hbm, table_hbm, out_hbm, idx_v, rows_v, sem):
    pltpu.sync_copy(y_hbm, idx_v)
    pltpu.async_copy(table_hbm.at[idx_v], rows_v, sem).wait()
    pltpu.sync_copy(rows_v, out_hbm)


_mesh = plsc.VectorSubcoreMesh(
    core_axis_name="c", subcore_axis_name="s", num_cores=1, num_subcores=1
)

_lookup = functools.partial(
    pl.kernel,
    mesh=_mesh,
    out_type=jax.ShapeDtypeStruct((_W,), jnp.int32),
    scratch_types=[
        pltpu.VMEM((_W,), jnp.int32),
        pltpu.VMEM((_W,), jnp.int32),
        pltpu.SemaphoreType.DMA,
    ],
)(_lookup_body)


def kernel(y, label_permutation):
    out_dtype = label_permutation.dtype
    table32 = label_permutation.astype(jnp.int32)
    y32 = jnp.full((_W,), y, jnp.int32)
    out = _lookup(y32, table32)
    return out[0].astype(out_dtype)


# speedup vs baseline: 1.0919x; 1.0013x over previous
import functools

import jax
import jax.numpy as jnp
from jax.experimental import pallas as pl
from jax.experimental.pallas import tpu as pltpu
from jax.experimental.pallas import tpu_sc as plsc

_W = 8


def _lookup_body(y_hbm, table_hbm, out_hbm, idx_v, rows_v, sem):
    @pl.when(jax.lax.axis_index("s") == 0)
    def _():
        pltpu.sync_copy(y_hbm, idx_v)
        pltpu.async_copy(table_hbm.at[idx_v], rows_v, sem).wait()
        pltpu.sync_copy(rows_v, out_hbm)


_mesh = plsc.VectorSubcoreMesh(
    core_axis_name="c", subcore_axis_name="s", num_cores=1, num_subcores=1
)

_lookup = functools.partial(
    pl.kernel,
    mesh=_mesh,
    out_type=jax.ShapeDtypeStruct((_W,), jnp.int32),
    scratch_types=[
        pltpu.VMEM((_W,), jnp.int32),
        pltpu.VMEM((_W,), jnp.int32),
        pltpu.SemaphoreType.DMA,
    ],
)(_lookup_body)


def kernel(y, label_permutation):
    out_dtype = label_permutation.dtype
    table32 = label_permutation.astype(jnp.int32)
    y32 = jnp.full((_W,), y, jnp.int32)
    out = _lookup(y32, table32)
    return out[0].astype(out_dtype)
